# baseline (speedup 1.0000x reference)
import functools

import jax
import jax.numpy as jnp
import numpy as np
from jax import lax
from jax.experimental import pallas as pl
from jax.experimental.pallas import tpu as pltpu
from jax.experimental.pallas import tpu_sc as plsc

_P = 16
_RATIO = 0.75

_KEEP_576 = (
    "010001000001011101000010010100010010111100011101110010100000111110001100"
    "000011100000000000010000001000000100000110001001010100010000000000000101"
    "100100000001100000000001000110000000000000100000001000000011000000000000"
    "010000110101010100000100101001001001110000110001000001000000010110011111"
    "000000000000001000000000000100000000100000010010100010001100000000000000"
    "000000010010000001000010010000100011010100010101110110001000000000100100"
    "000010000000000000000010001100000110001110000000010001010001010011000000"
    "011000000000100000001110001001000000000011011010000000000000010000100000"
)

_HB = 24
_WB = 24
_ZWIDTHS = (8, 4, 2, 1)


def _runs(row):
    out, c, n = [], 0, len(row)
    while c < n:
        if row[c]:
            c0 = c
            while c < n and row[c]:
                c += 1
            out.append((c0, c - c0))
        else:
            c += 1
    return out


def _build_tables():
    keep = np.array([ch == "1" for ch in _KEEP_576], dtype=bool)
    keep = keep.reshape(_HB, _WB)
    ztabs = {w: [] for w in _ZWIDTHS}
    k1, k2 = [[] for _ in range(_HB)], [[] for _ in range(_HB)]
    for i in range(_HB):
        for (c0, ln) in _runs(~keep[i]):
            c = c0
            rem = ln
            for w in _ZWIDTHS:
                while rem >= w:
                    ztabs[w].append(i * 32 + c)
                    c += w
                    rem -= w
        for (c0, ln) in _runs(keep[i]):
            if ln == 1:
                k1[i].append(i * 32 + c0)
            else:
                c = c0
                while c + 2 <= c0 + ln:
                    k2[i].append(i * 32 + c)
                    c += 2
                if c < c0 + ln:
                    k2[i].append(i * 32 + (c0 + ln - 2))
    return ztabs, k1, k2


def _csr(rows):
    ptr, flat = [0], []
    for r in rows:
        flat.extend(r)
        ptr.append(len(flat))
    return flat, ptr


def _sc_body(ntab, x_hbm, out_hbm, buf, zbuf, tab, k1ptr, k2ptr,
             sem_z, sem_in, sem_out):
    zt, k1, k1p, k2, k2p, npt = ntab
    nc = 2
    wid = lax.axis_index("s") * nc + lax.axis_index("c")
    base = wid * npt

    off = 0
    for w in _ZWIDTHS:
        for v in zt[w]:
            tab[off] = v
            off += 1
    for v in k1:
        tab[off] = v
        off += 1
    for v in k2:
        tab[off] = v
        off += 1
    for j in range(_HB + 1):
        k1ptr[j] = k1p[j]
        k2ptr[j] = k2p[j]

    def _zf(j, _):
        img = j // (_P * 8)
        rem = j % (_P * 8)
        zbuf[img, rem // 8, pl.ds((rem % 8) * 16, 16)] = jnp.zeros(
            (16,), jnp.float32)
        return _
    lax.fori_loop(0, npt * _P * 8, _zf, 0, unroll=8)

    off = 0
    for w in _ZWIDTHS:
        def _zscat(t, _, w=w, off=off):
            v = tab[t]
            i = v >> 5
            c0 = v & 31
            pltpu.make_async_copy(
                zbuf.at[:, :, pl.ds(0, w * _P)],
                out_hbm.at[pl.ds(base, npt), i, :, pl.ds(c0 * _P, w * _P)],
                sem_z).start()
            return _
        lax.fori_loop(off, off + len(zt[w]), _zscat, 0)
        off += len(zt[w])
    k1off = off
    k2off = off + len(k1)

    def _gath(t, _, w, toff):
        v = tab[t + toff]
        i = v >> 5
        c0 = v & 31
        pltpu.make_async_copy(
            x_hbm.at[pl.ds(base, npt), i, :, pl.ds(c0 * _P, w * _P)],
            buf.at[:, :, pl.ds(c0 * _P, w * _P)],
            sem_in).start()
        return _

    def _scat(t, _, w, toff):
        v = tab[t + toff]
        i = v >> 5
        c0 = v & 31
        pltpu.make_async_copy(
            buf.at[:, :, pl.ds(c0 * _P, w * _P)],
            out_hbm.at[pl.ds(base, npt), i, :, pl.ds(c0 * _P, w * _P)],
            sem_out).start()
        return _

    def _drain(n, w, sem):
        def _d(t, _):
            pltpu.make_async_copy(
                x_hbm.at[pl.ds(0, npt), 0, :, pl.ds(0, w * _P)],
                buf.at[:, :, pl.ds(0, w * _P)],
                sem).wait()
            return _
        lax.fori_loop(0, n, _d, 0)

    def _row(i, _):
        a1 = k1ptr[i]
        b1 = k1ptr[i + 1]
        a2 = k2ptr[i]
        b2 = k2ptr[i + 1]
        pa1 = jnp.where(i > 0, k1ptr[jnp.maximum(i - 1, 0)], a1)
        pa2 = jnp.where(i > 0, k2ptr[jnp.maximum(i - 1, 0)], a2)
        _drain(a1 - pa1, 1, sem_out)
        _drain(a2 - pa2, 2, sem_out)
        lax.fori_loop(a1, b1, functools.partial(_gath, w=1, toff=k1off), 0)
        lax.fori_loop(a2, b2, functools.partial(_gath, w=2, toff=k2off), 0)
        _drain(b1 - a1, 1, sem_in)
        _drain(b2 - a2, 2, sem_in)
        lax.fori_loop(a1, b1, functools.partial(_scat, w=1, toff=k1off), 0)
        lax.fori_loop(a2, b2, functools.partial(_scat, w=2, toff=k2off), 0)
        return _

    lax.fori_loop(0, _HB, _row, 0)

    nlast1 = k1p[_HB] - k1p[_HB - 1]
    nlast2 = k2p[_HB] - k2p[_HB - 1]
    _drain(nlast1, 1, sem_out)
    _drain(nlast2, 2, sem_out)
    for w in _ZWIDTHS:
        def _dz(t, _, w=w):
            pltpu.make_async_copy(
                x_hbm.at[pl.ds(0, npt), 0, :, pl.ds(0, w * _P)],
                buf.at[:, :, pl.ds(0, w * _P)],
                sem_z).wait()
            return _
        lax.fori_loop(0, len(zt[w]), _dz, 0)


def kernel(x):
    B, C, H, W = x.shape
    hb, wb = H // _P, W // _P
    assert (hb, wb) == (_HB, _WB)
    N = B * C

    zt, k1rows, k2rows = _build_tables()
    k1, k1p = _csr(k1rows)
    k2, k2p = _csr(k2rows)
    ntab = (zt, k1, k1p, k2, k2p, N // 32)
    nz = sum(len(v) for v in zt.values())
    tab_len = nz + len(k1) + len(k2)

    info = plsc.get_sparse_core_info()
    nw = info.num_cores * info.num_subcores
    assert nw == 32 and N % nw == 0

    xf = x.reshape(N, hb, _P, W)
    mesh = plsc.VectorSubcoreMesh(core_axis_name="c", subcore_axis_name="s")
    body = functools.partial(_sc_body, ntab)
    k = pl.kernel(
        body,
        mesh=mesh,
        compiler_params=pltpu.CompilerParams(use_tc_tiling_on_sc=False),
        out_type=jax.ShapeDtypeStruct((N, hb, _P, W), jnp.float32),
        scratch_types=[
            pltpu.VMEM((N // 32, _P, W), jnp.float32),
            pltpu.VMEM((N // 32, _P, 8 * _P), jnp.float32),
            pltpu.SMEM((tab_len,), jnp.int32),
            pltpu.SMEM((_HB + 1,), jnp.int32),
            pltpu.SMEM((_HB + 1,), jnp.int32),
            pltpu.SemaphoreType.DMA,
            pltpu.SemaphoreType.DMA,
            pltpu.SemaphoreType.DMA,
        ],
    )
    out = k(xf)
    return out.reshape(B, C, H, W)
